# Initial kernel scaffold; baseline (speedup 1.0000x reference)
import jax
import jax.numpy as jnp
from jax.experimental import pallas as pl


def kernel(image_features, text_features, taus_I, taus_T, s_I, s_T, b_I, b_T, z_I, z_T, p_I, p_T, group_info_I, group_info_T, image_ids, text_ids, epoch, max_epoch):
    raise NotImplementedError("write your pallas kernel here")


# speedup vs baseline: 2.3448x; 2.3448x over previous
import functools

import jax
import jax.numpy as jnp
from jax import lax
from jax.experimental import pallas as pl
from jax.experimental.pallas import tpu as pltpu
from jax.experimental.pallas import tpu_sc as plsc

B = 2048
D = 256
G = 8
GAMMA = 0.8
RHO_I = 0.1
RHO_T = 0.1
ETA_P = 0.01
LAMBADA = 0.5
EPS = 1e-10
GRAD_CLIP = 5.0

R = 256
NSTEP = B // R
BM1 = float(B - 1)

_HI = lax.Precision.HIGHEST
_f32 = jnp.float32
_i32 = jnp.int32


SC_CORES = 2
SC_SUBCORES = 16


@functools.cache
def _build_sc_gather():
    nw = SC_CORES * SC_SUBCORES
    bpw = B // nw
    mesh = plsc.VectorSubcoreMesh(core_axis_name="c", subcore_axis_name="s",
                                  num_cores=SC_CORES,
                                  num_subcores=SC_SUBCORES)
    out_type = ([jax.ShapeDtypeStruct((B,), _f32)] * 6
                + [jax.ShapeDtypeStruct((B,), _i32)] * 2)

    @functools.partial(
        pl.kernel, mesh=mesh, out_type=out_type,
        scratch_types=[pltpu.VMEM((bpw,), _i32),
                       pltpu.VMEM((bpw,), _f32),
                       pltpu.VMEM((bpw,), _i32),
                       pltpu.SemaphoreType.DMA],
    )
    def sc_gather(taus_i, taus_t, s_i, s_t, b_i, b_t, gi_i, gi_t,
                  ids_i, ids_t,
                  o_tau_i, o_tau_t, o_s_i, o_s_t, o_b_i, o_b_t,
                  o_gi_i, o_gi_t,
                  idx_v, fval_v, ival_v, sem):
        wid = lax.axis_index("s") * SC_CORES + lax.axis_index("c")
        sl = pl.ds(wid * bpw, bpw)
        for ids, ftabs, gtab, gout in (
                (ids_i, ((taus_i, o_tau_i), (s_i, o_s_i), (b_i, o_b_i)),
                 gi_i, o_gi_i),
                (ids_t, ((taus_t, o_tau_t), (s_t, o_s_t), (b_t, o_b_t)),
                 gi_t, o_gi_t)):
            pltpu.sync_copy(ids.at[sl], idx_v)
            for tbl, out in ftabs:
                pltpu.async_copy(tbl.at[idx_v], fval_v, sem).wait()
                pltpu.sync_copy(fval_v, out.at[sl])
            pltpu.async_copy(gtab.at[idx_v], ival_v, sem).wait()
            pltpu.sync_copy(ival_v, gout.at[sl])

    return sc_gather


def _sc_gather(*args):
    return _build_sc_gather()(*args)


def _pass1_body(img_ref, txt_ref, rs_ref, cs_ref, d_ref):
    i = pl.program_id(0)
    sim = lax.dot_general(img_ref[...], txt_ref[...],
                          (((1,), (1,)), ((), ())),
                          precision=_HI, preferred_element_type=_f32)
    rs_ref[0, :] = jnp.max(sim, axis=1)
    col = lax.broadcasted_iota(_i32, (R, B), 1)
    row = lax.broadcasted_iota(_i32, (R, B), 0)
    d_ref[0, :] = jnp.sum(jnp.where(col == row + i * R, sim, 0.0), axis=1)

    @pl.when(i == 0)
    def _():
        cs_ref[...] = jnp.full((1, B), -jnp.inf, _f32)

    cs_ref[0, :] = jnp.maximum(cs_ref[0, :], jnp.max(sim, axis=0))


def _pass1(img, txt):
    return pl.pallas_call(
        _pass1_body,
        grid=(NSTEP,),
        in_specs=[pl.BlockSpec((R, D), lambda i: (i, 0)),
                  pl.BlockSpec((B, D), lambda i: (0, 0))],
        out_specs=[pl.BlockSpec((1, R), lambda i: (0, i)),
                   pl.BlockSpec((1, B), lambda i: (0, 0)),
                   pl.BlockSpec((1, R), lambda i: (0, i))],
        out_shape=[jax.ShapeDtypeStruct((1, B), _f32),
                   jax.ShapeDtypeStruct((1, B), _f32),
                   jax.ShapeDtypeStruct((1, B), _f32)],
    )(img, txt)


def _pass2_body(ids_i_blk, ids_t_blk, ids_i, ids_t, rs_ref, cs_ref, d_ref,
                tau_i_ref, tau_t_ref, b_i_ref, b_t_ref,
                last_i_ref, last_t_ref, cb_i_ref, cb_t_ref):
    d = d_ref[0, :]
    kidx = lax.broadcasted_iota(_i32, (R, B), 1)
    for ids_blk_ref, ids_ref, mx_ref, tau_ref, bo_ref, last_ref, cb_ref in (
            (ids_i_blk, ids_i, rs_ref, tau_i_ref, b_i_ref,
             last_i_ref, cb_i_ref),
            (ids_t_blk, ids_t, cs_ref, tau_t_ref, b_t_ref,
             last_t_ref, cb_t_ref)):
        new_b = jnp.maximum((mx_ref[0, :] - d) / tau_ref[0, :], bo_ref[0, :])
        eq = ids_blk_ref[0, :][:, None] == ids_ref[0, :][None, :]
        last = jnp.max(jnp.where(eq, kidx, -1), axis=1)
        cb = jnp.sum(jnp.where(kidx == last[:, None], new_b[None, :], 0.0),
                     axis=1)
        last_ref[0, :] = last
        cb_ref[0, :] = cb


def _pass2(ids_i, ids_t, rs, cs, d, tau_i, tau_t, b_i, b_t):
    blk = pl.BlockSpec((1, R), lambda i: (0, i))
    full = pl.BlockSpec((1, B), lambda i: (0, 0))
    return pl.pallas_call(
        _pass2_body,
        grid=(NSTEP,),
        in_specs=[blk, blk, full, full, full, full, full, full, full, full,
                  full],
        out_specs=[blk, blk, blk, blk],
        out_shape=[jax.ShapeDtypeStruct((1, B), _i32),
                   jax.ShapeDtypeStruct((1, B), _i32),
                   jax.ShapeDtypeStruct((1, B), _f32),
                   jax.ShapeDtypeStruct((1, B), _f32)],
    )(ids_i, ids_t, ids_i, ids_t, rs, cs, d, tau_i, tau_t, b_i, b_t)


def _pass3_body(img_ref, txt_ref, d_blk_ref, tau_i_blk_ref, cb_i_blk_ref,
                d_ref, tau_t_ref, cb_t_ref,
                sum_i_ref, num_i_ref, sum_t_ref, num_t_ref):
    i = pl.program_id(0)
    sim = lax.dot_general(img_ref[...], txt_ref[...],
                          (((1,), (1,)), ((), ())),
                          precision=_HI, preferred_element_type=_f32)
    diff_i = sim - d_blk_ref[0, :][:, None]
    e_i = jnp.exp(diff_i / tau_i_blk_ref[0, :][:, None]
                  - cb_i_blk_ref[0, :][:, None])
    sum_i_ref[0, :] = jnp.sum(e_i, axis=1)
    num_i_ref[0, :] = jnp.sum(e_i * diff_i, axis=1)

    diff_t = sim - d_ref[0, :][None, :]
    e_t = jnp.exp(diff_t / tau_t_ref[0, :][None, :]
                  - cb_t_ref[0, :][None, :])

    @pl.when(i == 0)
    def _():
        sum_t_ref[...] = jnp.zeros((1, B), _f32)
        num_t_ref[...] = jnp.zeros((1, B), _f32)

    sum_t_ref[0, :] += jnp.sum(e_t, axis=0)
    num_t_ref[0, :] += jnp.sum(e_t * diff_t, axis=0)


def _pass3(img, txt, d, tau_i, cb_i, tau_t, cb_t):
    blk = pl.BlockSpec((1, R), lambda i: (0, i))
    full = pl.BlockSpec((1, B), lambda i: (0, 0))
    return pl.pallas_call(
        _pass3_body,
        grid=(NSTEP,),
        in_specs=[pl.BlockSpec((R, D), lambda i: (i, 0)),
                  pl.BlockSpec((B, D), lambda i: (0, 0)),
                  blk, blk, blk, full, full, full],
        out_specs=[blk, blk, full, full],
        out_shape=[jax.ShapeDtypeStruct((1, B), _f32)] * 4,
    )(img, txt, d, tau_i, cb_i, d, tau_t, cb_t)


def _pass4_body(sum_i_ref, num_i_ref, sum_t_ref, num_t_ref,
                cb_i_ref, cb_t_ref, tau_i_ref, tau_t_ref,
                so_i_ref, so_t_ref, bo_i_ref, bo_t_ref,
                gid_i_ref, gid_t_ref, last_i_ref, last_t_ref,
                p_i_ref, p_t_ref, z_i_ref, z_t_ref,
                loss_ref, po_i_ref, po_t_ref):
    kidx = lax.broadcasted_iota(_i32, (R, B), 1)
    grp = lax.broadcasted_iota(_i32, (G, B), 0)

    means = []
    updates = []
    for (sum_ref, num_ref, cb_ref, tau_ref, so_ref, bo_ref, gid_ref,
         last_ref, p_ref, z_ref, rho) in (
            (sum_i_ref, num_i_ref, cb_i_ref, tau_i_ref, so_i_ref, bo_i_ref,
             gid_i_ref, last_i_ref, p_i_ref, z_i_ref, RHO_I),
            (sum_t_ref, num_t_ref, cb_t_ref, tau_t_ref, so_t_ref, bo_t_ref,
             gid_t_ref, last_t_ref, p_t_ref, z_t_ref, RHO_T)):
        cb = cb_ref[0, :]
        g = sum_ref[0, :] / BM1
        s_vals = (1.0 - GAMMA) * so_ref[0, :] * jnp.exp(bo_ref[0, :] - cb) \
            + GAMMA * g
        chunks = []
        for s in range(NSTEP):
            last_blk = last_ref[0, pl.ds(s * R, R)]
            oh = kidx == last_blk[:, None]
            chunks.append(jnp.sum(jnp.where(oh, s_vals[None, :], 0.0),
                                  axis=1))
        s_b = jnp.concatenate(chunks)

        gid = gid_ref[0, :]
        p = p_ref[0, :]
        oh_g = grp == gid[None, :]
        gw = G * jnp.sum(jnp.where(oh_g, p[:, None], 0.0), axis=0)
        loss = gw * num_ref[0, :] / BM1 / (s_b + EPS)
        means.append(jnp.mean(loss))

        f = tau_ref[0, :] * (jnp.log(s_b) + cb + rho)
        counts = jnp.sum(oh_g.astype(_f32), axis=1)
        gsum = jnp.sum(jnp.where(oh_g, f[None, :], 0.0), axis=1)
        grad = gsum / jnp.maximum(counts, 1.0)
        z = (1.0 - GAMMA) * z_ref[0, :] + GAMMA * grad
        ghp = -LAMBADA * jnp.log(p + EPS) - LAMBADA
        new_p = p * jnp.exp(2.0 * ETA_P
                            * jnp.clip(z + ghp, -GRAD_CLIP, GRAD_CLIP))
        updates.append(new_p / jnp.sum(new_p))

    loss_ref[0, 0] = means[0] + means[1]
    po_i_ref[0, :] = updates[0]
    po_t_ref[0, :] = updates[1]


def _pass4(sum_i, num_i, sum_t, num_t, cb_i, cb_t, tau_i, tau_t,
           so_i, so_t, bo_i, bo_t, gid_i, gid_t, last_i, last_t,
           p_i, p_t, z_i, z_t):
    return pl.pallas_call(
        _pass4_body,
        out_shape=[jax.ShapeDtypeStruct((1, 1), _f32),
                   jax.ShapeDtypeStruct((1, G), _f32),
                   jax.ShapeDtypeStruct((1, G), _f32)],
        out_specs=[pl.BlockSpec(memory_space=pltpu.SMEM),
                   pl.BlockSpec((1, G), lambda: (0, 0)),
                   pl.BlockSpec((1, G), lambda: (0, 0))],
    )(sum_i, num_i, sum_t, num_t, cb_i, cb_t, tau_i, tau_t,
      so_i, so_t, bo_i, bo_t, gid_i, gid_t, last_i, last_t,
      p_i, p_t, z_i, z_t)


def kernel(image_features, text_features, taus_I, taus_T, s_I, s_T, b_I, b_T,
           z_I, z_T, p_I, p_T, group_info_I, group_info_T,
           image_ids, text_ids, epoch, max_epoch):
    tau_i, tau_t, so_i, so_t, bo_i, bo_t, gid_i, gid_t = _sc_gather(
        taus_I, taus_T, s_I, s_T, b_I, b_T,
        group_info_I.astype(_i32), group_info_T.astype(_i32),
        image_ids.astype(_i32), text_ids.astype(_i32))

    row = lambda v: v.reshape(1, -1)
    rs, cs, d = _pass1(image_features, text_features)
    last_i, last_t, cb_i, cb_t = _pass2(
        row(image_ids.astype(_i32)), row(text_ids.astype(_i32)),
        rs, cs, d, row(tau_i), row(tau_t), row(bo_i), row(bo_t))
    sum_i, num_i, sum_t, num_t = _pass3(
        image_features, text_features, d, row(tau_i), cb_i, row(tau_t), cb_t)
    loss, p_i_new, p_t_new = _pass4(
        sum_i, num_i, sum_t, num_t, cb_i, cb_t, row(tau_i), row(tau_t),
        row(so_i), row(so_t), row(bo_i), row(bo_t),
        row(gid_i), row(gid_t), last_i, last_t,
        row(p_I), row(p_T), row(z_I), row(z_T))
    return loss[0, 0], p_i_new[0, :], p_t_new[0, :]


# speedup vs baseline: 2.6217x; 1.1181x over previous
import functools

import jax
import jax.numpy as jnp
from jax import lax
from jax.experimental import pallas as pl
from jax.experimental.pallas import tpu as pltpu
from jax.experimental.pallas import tpu_sc as plsc

B = 2048
D = 256
G = 8
GAMMA = 0.8
RHO_I = 0.1
RHO_T = 0.1
ETA_P = 0.01
LAMBADA = 0.5
EPS = 1e-10
GRAD_CLIP = 5.0

R = 256
NSTEP = B // R
BM1 = float(B - 1)

_HI = lax.Precision.HIGHEST
_f32 = jnp.float32
_i32 = jnp.int32


SC_CORES = 2
SC_SUBCORES = 16


@functools.cache
def _build_sc_gather():
    nw = SC_CORES * SC_SUBCORES
    bpw = B // nw
    mesh = plsc.VectorSubcoreMesh(core_axis_name="c", subcore_axis_name="s",
                                  num_cores=SC_CORES,
                                  num_subcores=SC_SUBCORES)
    out_type = ([jax.ShapeDtypeStruct((B,), _f32)] * 6
                + [jax.ShapeDtypeStruct((B,), _i32)] * 2)

    @functools.partial(
        pl.kernel, mesh=mesh, out_type=out_type,
        scratch_types=[pltpu.VMEM((bpw,), _i32),
                       pltpu.VMEM((bpw,), _f32),
                       pltpu.VMEM((bpw,), _i32),
                       pltpu.SemaphoreType.DMA],
    )
    def sc_gather(taus_i, taus_t, s_i, s_t, b_i, b_t, gi_i, gi_t,
                  ids_i, ids_t,
                  o_tau_i, o_tau_t, o_s_i, o_s_t, o_b_i, o_b_t,
                  o_gi_i, o_gi_t,
                  idx_v, fval_v, ival_v, sem):
        wid = lax.axis_index("s") * SC_CORES + lax.axis_index("c")
        sl = pl.ds(wid * bpw, bpw)
        for ids, ftabs, gtab, gout in (
                (ids_i, ((taus_i, o_tau_i), (s_i, o_s_i), (b_i, o_b_i)),
                 gi_i, o_gi_i),
                (ids_t, ((taus_t, o_tau_t), (s_t, o_s_t), (b_t, o_b_t)),
                 gi_t, o_gi_t)):
            pltpu.sync_copy(ids.at[sl], idx_v)
            for tbl, out in ftabs:
                pltpu.async_copy(tbl.at[idx_v], fval_v, sem).wait()
                pltpu.sync_copy(fval_v, out.at[sl])
            pltpu.async_copy(gtab.at[idx_v], ival_v, sem).wait()
            pltpu.sync_copy(ival_v, gout.at[sl])

    return sc_gather


def _sc_gather(*args):
    return _build_sc_gather()(*args)


def _fused_body(ids_i_ref, ids_t_ref, img_ref, txt_ref,
                tau_i_ref, tau_t_ref, so_i_ref, so_t_ref,
                bo_i_ref, bo_t_ref, gid_i_ref, gid_t_ref,
                p_i_ref, p_t_ref, z_i_ref, z_t_ref,
                loss_ref, po_i_ref, po_t_ref,
                sim_ref, rs_ref, cs_ref, d_ref):
    i = pl.program_id(0)

    @pl.when(i < NSTEP)
    def _():
        sim = lax.dot_general(img_ref[...], txt_ref[...],
                              (((1,), (1,)), ((), ())),
                              precision=_HI, preferred_element_type=_f32)
        sim_ref[pl.ds(i * R, R), :] = sim
        rs_ref[0, pl.ds(i * R, R)] = jnp.max(sim, axis=1)
        col = lax.broadcasted_iota(_i32, (R, B), 1)
        row = lax.broadcasted_iota(_i32, (R, B), 0)
        d_ref[0, pl.ds(i * R, R)] = jnp.sum(
            jnp.where(col == row + i * R, sim, 0.0), axis=1)

        @pl.when(i == 0)
        def _():
            cs_ref[...] = jnp.full((1, B), -jnp.inf, _f32)

        cs_ref[0, :] = jnp.maximum(cs_ref[0, :], jnp.max(sim, axis=0))

    @pl.when(i == NSTEP)
    def _():
        kidx = lax.broadcasted_iota(_i32, (R, B), 1)
        d = d_ref[0, :]

        def resolve(ids_ref, mx, tau, bo):
            new_b = jnp.maximum((mx - d) / tau, bo)
            last_c, cb_c = [], []
            for s in range(NSTEP):
                ids_blk = ids_ref[0, pl.ds(s * R, R)]
                eq = ids_blk[:, None] == ids_ref[0, :][None, :]
                last = jnp.max(jnp.where(eq, kidx, -1), axis=1)
                cb = jnp.sum(jnp.where(kidx == last[:, None],
                                       new_b[None, :], 0.0), axis=1)
                last_c.append(last)
                cb_c.append(cb)
            return jnp.concatenate(last_c), jnp.concatenate(cb_c)

        tau_i = tau_i_ref[0, :]
        tau_t = tau_t_ref[0, :]
        last_i, cb_i = resolve(ids_i_ref, rs_ref[0, :], tau_i, bo_i_ref[0, :])
        last_t, cb_t = resolve(ids_t_ref, cs_ref[0, :], tau_t, bo_t_ref[0, :])

        sum_i_c, num_i_c = [], []
        sum_t = jnp.zeros((B,), _f32)
        num_t = jnp.zeros((B,), _f32)
        for s in range(NSTEP):
            lo, hi = s * R, (s + 1) * R
            sim = sim_ref[pl.ds(s * R, R), :]
            diff_i = sim - d[lo:hi][:, None]
            e_i = jnp.exp(diff_i / tau_i[lo:hi][:, None]
                          - cb_i[lo:hi][:, None])
            sum_i_c.append(jnp.sum(e_i, axis=1))
            num_i_c.append(jnp.sum(e_i * diff_i, axis=1))
            diff_t = sim - d[None, :]
            e_t = jnp.exp(diff_t / tau_t[None, :] - cb_t[None, :])
            sum_t += jnp.sum(e_t, axis=0)
            num_t += jnp.sum(e_t * diff_t, axis=0)
        sum_i = jnp.concatenate(sum_i_c)
        num_i = jnp.concatenate(num_i_c)

        grp = lax.broadcasted_iota(_i32, (G, B), 0)
        means = []
        updates = []
        for (sm, num, cb, last, tau, so_ref, bo_ref, gid_ref, p_ref, z_ref,
             rho) in (
                (sum_i, num_i, cb_i, last_i, tau_i, so_i_ref, bo_i_ref,
                 gid_i_ref, p_i_ref, z_i_ref, RHO_I),
                (sum_t, num_t, cb_t, last_t, tau_t, so_t_ref, bo_t_ref,
                 gid_t_ref, p_t_ref, z_t_ref, RHO_T)):
            g = sm / BM1
            s_vals = (1.0 - GAMMA) * so_ref[0, :] * jnp.exp(bo_ref[0, :] - cb) \
                + GAMMA * g
            chunks = []
            for s in range(NSTEP):
                oh = kidx == last[s * R:(s + 1) * R][:, None]
                chunks.append(jnp.sum(jnp.where(oh, s_vals[None, :], 0.0),
                                      axis=1))
            s_b = jnp.concatenate(chunks)

            gid = gid_ref[0, :]
            p = p_ref[0, :]
            oh_g = grp == gid[None, :]
            gw = G * jnp.sum(jnp.where(oh_g, p[:, None], 0.0), axis=0)
            loss = gw * num / BM1 / (s_b + EPS)
            means.append(jnp.mean(loss))

            f = tau * (jnp.log(s_b) + cb + rho)
            counts = jnp.sum(oh_g.astype(_f32), axis=1)
            gsum = jnp.sum(jnp.where(oh_g, f[None, :], 0.0), axis=1)
            grad = gsum / jnp.maximum(counts, 1.0)
            z = (1.0 - GAMMA) * z_ref[0, :] + GAMMA * grad
            ghp = -LAMBADA * jnp.log(p + EPS) - LAMBADA
            new_p = p * jnp.exp(2.0 * ETA_P
                                * jnp.clip(z + ghp, -GRAD_CLIP, GRAD_CLIP))
            updates.append(new_p / jnp.sum(new_p))

        loss_ref[0, 0] = means[0] + means[1]
        po_i_ref[0, :] = updates[0]
        po_t_ref[0, :] = updates[1]


def _fused(ids_i, ids_t, img, txt, tau_i, tau_t, so_i, so_t, bo_i, bo_t,
           gid_i, gid_t, p_i, p_t, z_i, z_t):
    full = pl.BlockSpec((1, B), lambda i: (0, 0))
    small = pl.BlockSpec((1, G), lambda i: (0, 0))
    return pl.pallas_call(
        _fused_body,
        grid=(NSTEP + 1,),
        in_specs=[full, full,
                  pl.BlockSpec((R, D), lambda i: (jnp.minimum(i, NSTEP - 1),
                                                  0)),
                  pl.BlockSpec((B, D), lambda i: (0, 0)),
                  full, full, full, full, full, full, full, full,
                  small, small, small, small],
        out_specs=[pl.BlockSpec(memory_space=pltpu.SMEM),
                   pl.BlockSpec((1, G), lambda i: (0, 0)),
                   pl.BlockSpec((1, G), lambda i: (0, 0))],
        out_shape=[jax.ShapeDtypeStruct((1, 1), _f32),
                   jax.ShapeDtypeStruct((1, G), _f32),
                   jax.ShapeDtypeStruct((1, G), _f32)],
        scratch_shapes=[pltpu.VMEM((B, B), _f32),
                        pltpu.VMEM((1, B), _f32),
                        pltpu.VMEM((1, B), _f32),
                        pltpu.VMEM((1, B), _f32)],
    )(ids_i, ids_t, img, txt, tau_i, tau_t, so_i, so_t, bo_i, bo_t,
      gid_i, gid_t, p_i, p_t, z_i, z_t)


def kernel(image_features, text_features, taus_I, taus_T, s_I, s_T, b_I, b_T,
           z_I, z_T, p_I, p_T, group_info_I, group_info_T,
           image_ids, text_ids, epoch, max_epoch):
    tau_i, tau_t, so_i, so_t, bo_i, bo_t, gid_i, gid_t = _sc_gather(
        taus_I, taus_T, s_I, s_T, b_I, b_T,
        group_info_I.astype(_i32), group_info_T.astype(_i32),
        image_ids.astype(_i32), text_ids.astype(_i32))

    row = lambda v: v.reshape(1, -1)
    loss, p_i_new, p_t_new = _fused(
        row(image_ids.astype(_i32)), row(text_ids.astype(_i32)),
        image_features, text_features,
        row(tau_i), row(tau_t), row(so_i), row(so_t),
        row(bo_i), row(bo_t), row(gid_i), row(gid_t),
        row(p_I), row(p_T), row(z_I), row(z_T))
    return loss[0, 0], p_i_new[0, :], p_t_new[0, :]


# speedup vs baseline: 2.8467x; 1.0858x over previous
import functools

import jax
import jax.numpy as jnp
from jax import lax
from jax.experimental import pallas as pl
from jax.experimental.pallas import tpu as pltpu
from jax.experimental.pallas import tpu_sc as plsc

B = 2048
D = 256
G = 8
GAMMA = 0.8
RHO_I = 0.1
RHO_T = 0.1
ETA_P = 0.01
LAMBADA = 0.5
EPS = 1e-10
GRAD_CLIP = 5.0

R = 256
NSTEP = B // R
BM1 = float(B - 1)

_HI = lax.Precision.HIGHEST
_f32 = jnp.float32
_i32 = jnp.int32


SC_CORES = 2
SC_SUBCORES = 16


@functools.cache
def _build_sc_gather():
    nw = SC_CORES * SC_SUBCORES
    bpw = B // nw
    mesh = plsc.VectorSubcoreMesh(core_axis_name="c", subcore_axis_name="s",
                                  num_cores=SC_CORES,
                                  num_subcores=SC_SUBCORES)
    out_type = ([jax.ShapeDtypeStruct((B,), _f32)] * 6
                + [jax.ShapeDtypeStruct((B,), _i32)] * 2)

    @functools.partial(
        pl.kernel, mesh=mesh, out_type=out_type,
        scratch_types=[pltpu.VMEM((bpw,), _i32),
                       pltpu.VMEM((bpw,), _i32)]
        + [pltpu.VMEM((bpw,), _f32)] * 6
        + [pltpu.VMEM((bpw,), _i32)] * 2
        + [pltpu.SemaphoreType.DMA] * 8,
    )
    def sc_gather(taus_i, taus_t, s_i, s_t, b_i, b_t, gi_i, gi_t,
                  ids_i, ids_t,
                  o_tau_i, o_tau_t, o_s_i, o_s_t, o_b_i, o_b_t,
                  o_gi_i, o_gi_t,
                  idx_i_v, idx_t_v, v0, v1, v2, v3, v4, v5, g0, g1,
                  s0, s1, s2, s3, s4, s5, s6, s7):
        wid = lax.axis_index("s") * SC_CORES + lax.axis_index("c")
        sl = pl.ds(wid * bpw, bpw)
        pltpu.sync_copy(ids_i.at[sl], idx_i_v)
        pltpu.sync_copy(ids_t.at[sl], idx_t_v)
        plan = ((taus_i, idx_i_v, v0, s0, o_tau_i),
                (s_i, idx_i_v, v1, s1, o_s_i),
                (b_i, idx_i_v, v2, s2, o_b_i),
                (gi_i, idx_i_v, g0, s6, o_gi_i),
                (taus_t, idx_t_v, v3, s3, o_tau_t),
                (s_t, idx_t_v, v4, s4, o_s_t),
                (b_t, idx_t_v, v5, s5, o_b_t),
                (gi_t, idx_t_v, g1, s7, o_gi_t))
        cps = [pltpu.async_copy(tbl.at[idx], dst, sem)
               for tbl, idx, dst, sem, _ in plan]
        for cp, (_, _, dst, _, out) in zip(cps, plan):
            cp.wait()
            pltpu.sync_copy(dst, out.at[sl])

    return sc_gather


def _sc_gather(*args):
    return _build_sc_gather()(*args)


def _fused_body(ids_i_ref, ids_t_ref, img_ref, txt_ref,
                tau_i_ref, tau_t_ref, so_i_ref, so_t_ref,
                bo_i_ref, bo_t_ref, gid_i_ref, gid_t_ref,
                p_i_ref, p_t_ref, z_i_ref, z_t_ref,
                loss_ref, po_i_ref, po_t_ref,
                sim_ref, rs_ref, cs_ref, d_ref):
    i = pl.program_id(0)

    @pl.when(i < NSTEP)
    def _():
        sim = lax.dot_general(img_ref[...], txt_ref[...],
                              (((1,), (1,)), ((), ())),
                              precision=_HI, preferred_element_type=_f32)
        sim_ref[pl.ds(i * R, R), :] = sim
        rs_ref[0, pl.ds(i * R, R)] = jnp.max(sim, axis=1)
        col = lax.broadcasted_iota(_i32, (R, B), 1)
        row = lax.broadcasted_iota(_i32, (R, B), 0)
        d_ref[0, pl.ds(i * R, R)] = jnp.sum(
            jnp.where(col == row + i * R, sim, 0.0), axis=1)

        @pl.when(i == 0)
        def _():
            cs_ref[...] = jnp.full((1, B), -jnp.inf, _f32)

        cs_ref[0, :] = jnp.maximum(cs_ref[0, :], jnp.max(sim, axis=0))

    @pl.when(i == NSTEP)
    def _():
        kidx = lax.broadcasted_iota(_i32, (R, B), 1)
        d = d_ref[0, :]
        rtau_i = 1.0 / tau_i_ref[0, :]
        rtau_t = 1.0 / tau_t_ref[0, :]

        def resolve(ids_ref, mx, rtau, bo):
            new_b = jnp.maximum((mx - d) * rtau, bo)
            last_c, cb_c = [], []
            for s in range(NSTEP):
                ids_blk = ids_ref[0, pl.ds(s * R, R)]
                eq = ids_blk[:, None] == ids_ref[0, :][None, :]
                last = jnp.max(jnp.where(eq, kidx, -1), axis=1)
                cb = jnp.sum(jnp.where(kidx == last[:, None],
                                       new_b[None, :], 0.0), axis=1)
                last_c.append(last)
                cb_c.append(cb)
            return jnp.concatenate(last_c), jnp.concatenate(cb_c)

        last_i, cb_i = resolve(ids_i_ref, rs_ref[0, :], rtau_i,
                               bo_i_ref[0, :])
        last_t, cb_t = resolve(ids_t_ref, cs_ref[0, :], rtau_t,
                               bo_t_ref[0, :])

        b_co_i = d * rtau_i + cb_i
        b_co_t = d * rtau_t + cb_t
        sum_i_c, es_i_c = [], []
        sum_t = jnp.zeros((B,), _f32)
        es_t = jnp.zeros((B,), _f32)
        for s in range(NSTEP):
            lo, hi = s * R, (s + 1) * R
            sim = sim_ref[pl.ds(s * R, R), :]
            e_i = jnp.exp(sim * rtau_i[lo:hi][:, None]
                          - b_co_i[lo:hi][:, None])
            sum_i_c.append(jnp.sum(e_i, axis=1))
            es_i_c.append(jnp.sum(e_i * sim, axis=1))
            e_t = jnp.exp(sim * rtau_t[None, :] - b_co_t[None, :])
            sum_t += jnp.sum(e_t, axis=0)
            es_t += jnp.sum(e_t * sim, axis=0)
        sum_i = jnp.concatenate(sum_i_c)
        num_i = jnp.concatenate(es_i_c) - d * sum_i
        num_t = es_t - d * sum_t

        grp = lax.broadcasted_iota(_i32, (G, B), 0)
        means = []
        updates = []
        for (sm, num, cb, last, tau, so_ref, bo_ref, gid_ref, p_ref, z_ref,
             rho) in (
                (sum_i, num_i, cb_i, last_i, tau_i_ref[0, :], so_i_ref,
                 bo_i_ref, gid_i_ref, p_i_ref, z_i_ref, RHO_I),
                (sum_t, num_t, cb_t, last_t, tau_t_ref[0, :], so_t_ref,
                 bo_t_ref, gid_t_ref, p_t_ref, z_t_ref, RHO_T)):
            g = sm / BM1
            s_vals = (1.0 - GAMMA) * so_ref[0, :] * jnp.exp(bo_ref[0, :] - cb) \
                + GAMMA * g
            chunks = []
            for s in range(NSTEP):
                oh = kidx == last[s * R:(s + 1) * R][:, None]
                chunks.append(jnp.sum(jnp.where(oh, s_vals[None, :], 0.0),
                                      axis=1))
            s_b = jnp.concatenate(chunks)

            gid = gid_ref[0, :]
            p = p_ref[0, :]
            oh_g = grp == gid[None, :]
            gw = G * jnp.sum(jnp.where(oh_g, p[:, None], 0.0), axis=0)
            loss = gw * num / BM1 / (s_b + EPS)
            means.append(jnp.mean(loss))

            f = tau * (jnp.log(s_b) + cb + rho)
            counts = jnp.sum(oh_g.astype(_f32), axis=1)
            gsum = jnp.sum(jnp.where(oh_g, f[None, :], 0.0), axis=1)
            grad = gsum / jnp.maximum(counts, 1.0)
            z = (1.0 - GAMMA) * z_ref[0, :] + GAMMA * grad
            ghp = -LAMBADA * jnp.log(p + EPS) - LAMBADA
            new_p = p * jnp.exp(2.0 * ETA_P
                                * jnp.clip(z + ghp, -GRAD_CLIP, GRAD_CLIP))
            updates.append(new_p / jnp.sum(new_p))

        loss_ref[0, 0] = means[0] + means[1]
        po_i_ref[0, :] = updates[0]
        po_t_ref[0, :] = updates[1]


def _fused(ids_i, ids_t, img, txt, tau_i, tau_t, so_i, so_t, bo_i, bo_t,
           gid_i, gid_t, p_i, p_t, z_i, z_t):
    full = pl.BlockSpec((1, B), lambda i: (0, 0))
    small = pl.BlockSpec((1, G), lambda i: (0, 0))
    return pl.pallas_call(
        _fused_body,
        grid=(NSTEP + 1,),
        in_specs=[full, full,
                  pl.BlockSpec((R, D), lambda i: (jnp.minimum(i, NSTEP - 1),
                                                  0)),
                  pl.BlockSpec((B, D), lambda i: (0, 0)),
                  full, full, full, full, full, full, full, full,
                  small, small, small, small],
        out_specs=[pl.BlockSpec(memory_space=pltpu.SMEM),
                   pl.BlockSpec((1, G), lambda i: (0, 0)),
                   pl.BlockSpec((1, G), lambda i: (0, 0))],
        out_shape=[jax.ShapeDtypeStruct((1, 1), _f32),
                   jax.ShapeDtypeStruct((1, G), _f32),
                   jax.ShapeDtypeStruct((1, G), _f32)],
        scratch_shapes=[pltpu.VMEM((B, B), _f32),
                        pltpu.VMEM((1, B), _f32),
                        pltpu.VMEM((1, B), _f32),
                        pltpu.VMEM((1, B), _f32)],
    )(ids_i, ids_t, img, txt, tau_i, tau_t, so_i, so_t, bo_i, bo_t,
      gid_i, gid_t, p_i, p_t, z_i, z_t)


def kernel(image_features, text_features, taus_I, taus_T, s_I, s_T, b_I, b_T,
           z_I, z_T, p_I, p_T, group_info_I, group_info_T,
           image_ids, text_ids, epoch, max_epoch):
    tau_i, tau_t, so_i, so_t, bo_i, bo_t, gid_i, gid_t = _sc_gather(
        taus_I, taus_T, s_I, s_T, b_I, b_T,
        group_info_I.astype(_i32), group_info_T.astype(_i32),
        image_ids.astype(_i32), text_ids.astype(_i32))

    row = lambda v: v.reshape(1, -1)
    loss, p_i_new, p_t_new = _fused(
        row(image_ids.astype(_i32)), row(text_ids.astype(_i32)),
        image_features, text_features,
        row(tau_i), row(tau_t), row(so_i), row(so_t),
        row(bo_i), row(bo_t), row(gid_i), row(gid_t),
        row(p_I), row(p_T), row(z_I), row(z_T))
    return loss[0, 0], p_i_new[0, :], p_t_new[0, :]


# speedup vs baseline: 3.3363x; 1.1720x over previous
import functools

import jax
import jax.numpy as jnp
from jax import lax
from jax.experimental import pallas as pl
from jax.experimental.pallas import tpu as pltpu
from jax.experimental.pallas import tpu_sc as plsc

B = 2048
D = 256
G = 8
GAMMA = 0.8
RHO_I = 0.1
RHO_T = 0.1
ETA_P = 0.01
LAMBADA = 0.5
EPS = 1e-10
GRAD_CLIP = 5.0

R = 256
NSTEP = B // R
BM1 = float(B - 1)

_HI = lax.Precision.HIGHEST
_f32 = jnp.float32
_i32 = jnp.int32


SC_CORES = 2
SC_SUBCORES = 16


@functools.cache
def _build_sc_gather():
    nw = SC_CORES * SC_SUBCORES
    bpw = B // nw
    mesh = plsc.VectorSubcoreMesh(core_axis_name="c", subcore_axis_name="s",
                                  num_cores=SC_CORES,
                                  num_subcores=SC_SUBCORES)
    out_type = ([jax.ShapeDtypeStruct((B,), _f32)] * 6
                + [jax.ShapeDtypeStruct((B,), _i32)] * 2)

    @functools.partial(
        pl.kernel, mesh=mesh, out_type=out_type,
        scratch_types=[pltpu.VMEM((bpw,), _i32),
                       pltpu.VMEM((bpw,), _i32)]
        + [pltpu.VMEM((bpw,), _f32)] * 6
        + [pltpu.VMEM((bpw,), _i32)] * 2
        + [pltpu.SemaphoreType.DMA] * 8,
    )
    def sc_gather(taus_i, taus_t, s_i, s_t, b_i, b_t, gi_i, gi_t,
                  ids_i, ids_t,
                  o_tau_i, o_tau_t, o_s_i, o_s_t, o_b_i, o_b_t,
                  o_gi_i, o_gi_t,
                  idx_i_v, idx_t_v, v0, v1, v2, v3, v4, v5, g0, g1,
                  s0, s1, s2, s3, s4, s5, s6, s7):
        wid = lax.axis_index("s") * SC_CORES + lax.axis_index("c")
        sl = pl.ds(wid * bpw, bpw)
        pltpu.sync_copy(ids_i.at[sl], idx_i_v)
        pltpu.sync_copy(ids_t.at[sl], idx_t_v)
        plan = ((taus_i, idx_i_v, v0, s0, o_tau_i),
                (s_i, idx_i_v, v1, s1, o_s_i),
                (b_i, idx_i_v, v2, s2, o_b_i),
                (gi_i, idx_i_v, g0, s6, o_gi_i),
                (taus_t, idx_t_v, v3, s3, o_tau_t),
                (s_t, idx_t_v, v4, s4, o_s_t),
                (b_t, idx_t_v, v5, s5, o_b_t),
                (gi_t, idx_t_v, g1, s7, o_gi_t))
        cps = [pltpu.async_copy(tbl.at[idx], dst, sem)
               for tbl, idx, dst, sem, _ in plan]
        for cp, (_, _, dst, _, out) in zip(cps, plan):
            cp.wait()
            pltpu.sync_copy(dst, out.at[sl])

    return sc_gather


def _sc_gather(*args):
    return _build_sc_gather()(*args)


def _fused_body(ids_i_ref, ids_t_ref, img_ref, txt_ref,
                tau_i_ref, tau_t_ref, so_i_ref, so_t_ref,
                bo_i_ref, bo_t_ref, gid_i_ref, gid_t_ref,
                p_i_ref, p_t_ref, z_i_ref, z_t_ref,
                loss_ref, po_i_ref, po_t_ref,
                sim_ref, rs_ref, cs_ref, d_ref):
    i = pl.program_id(0)

    @pl.when(i < NSTEP)
    def _():
        sim = lax.dot_general(img_ref[...], txt_ref[...],
                              (((1,), (1,)), ((), ())),
                              preferred_element_type=_f32)
        sim_ref[pl.ds(i * R, R), :] = sim
        rs_ref[0, pl.ds(i * R, R)] = jnp.max(sim, axis=1)
        col = lax.broadcasted_iota(_i32, (R, B), 1)
        row = lax.broadcasted_iota(_i32, (R, B), 0)
        d_ref[0, pl.ds(i * R, R)] = jnp.sum(
            jnp.where(col == row + i * R, sim, 0.0), axis=1)

        @pl.when(i == 0)
        def _():
            cs_ref[...] = jnp.full((1, B), -jnp.inf, _f32)

        cs_ref[0, :] = jnp.maximum(cs_ref[0, :], jnp.max(sim, axis=0))

    @pl.when(i == NSTEP)
    def _():
        kidx = lax.broadcasted_iota(_i32, (R, B), 1)
        d = d_ref[0, :]
        rtau_i = 1.0 / tau_i_ref[0, :]
        rtau_t = 1.0 / tau_t_ref[0, :]

        def resolve(ids_ref, mx, rtau, bo):
            new_b = jnp.maximum((mx - d) * rtau, bo)
            last_c, cb_c = [], []
            for s in range(NSTEP):
                ids_blk = ids_ref[0, pl.ds(s * R, R)]
                eq = ids_blk[:, None] == ids_ref[0, :][None, :]
                last = jnp.max(jnp.where(eq, kidx, -1), axis=1)
                cb = jnp.sum(jnp.where(kidx == last[:, None],
                                       new_b[None, :], 0.0), axis=1)
                last_c.append(last)
                cb_c.append(cb)
            return jnp.concatenate(last_c), jnp.concatenate(cb_c)

        last_i, cb_i = resolve(ids_i_ref, rs_ref[0, :], rtau_i,
                               bo_i_ref[0, :])
        last_t, cb_t = resolve(ids_t_ref, cs_ref[0, :], rtau_t,
                               bo_t_ref[0, :])

        b_co_i = d * rtau_i + cb_i
        b_co_t = d * rtau_t + cb_t
        sum_i_c, es_i_c = [], []
        sum_t = jnp.zeros((B,), _f32)
        es_t = jnp.zeros((B,), _f32)
        for s in range(NSTEP):
            lo, hi = s * R, (s + 1) * R
            sim = sim_ref[pl.ds(s * R, R), :]
            e_i = jnp.exp(sim * rtau_i[lo:hi][:, None]
                          - b_co_i[lo:hi][:, None])
            sum_i_c.append(jnp.sum(e_i, axis=1))
            es_i_c.append(jnp.sum(e_i * sim, axis=1))
            e_t = jnp.exp(sim * rtau_t[None, :] - b_co_t[None, :])
            sum_t += jnp.sum(e_t, axis=0)
            es_t += jnp.sum(e_t * sim, axis=0)
        sum_i = jnp.concatenate(sum_i_c)
        num_i = jnp.concatenate(es_i_c) - d * sum_i
        num_t = es_t - d * sum_t

        grp = lax.broadcasted_iota(_i32, (G, B), 0)
        means = []
        updates = []
        for (sm, num, cb, last, tau, so_ref, bo_ref, gid_ref, p_ref, z_ref,
             rho) in (
                (sum_i, num_i, cb_i, last_i, tau_i_ref[0, :], so_i_ref,
                 bo_i_ref, gid_i_ref, p_i_ref, z_i_ref, RHO_I),
                (sum_t, num_t, cb_t, last_t, tau_t_ref[0, :], so_t_ref,
                 bo_t_ref, gid_t_ref, p_t_ref, z_t_ref, RHO_T)):
            g = sm / BM1
            s_vals = (1.0 - GAMMA) * so_ref[0, :] * jnp.exp(bo_ref[0, :] - cb) \
                + GAMMA * g
            chunks = []
            for s in range(NSTEP):
                oh = kidx == last[s * R:(s + 1) * R][:, None]
                chunks.append(jnp.sum(jnp.where(oh, s_vals[None, :], 0.0),
                                      axis=1))
            s_b = jnp.concatenate(chunks)

            gid = gid_ref[0, :]
            p = p_ref[0, :]
            oh_g = grp == gid[None, :]
            gw = G * jnp.sum(jnp.where(oh_g, p[:, None], 0.0), axis=0)
            loss = gw * num / BM1 / (s_b + EPS)
            means.append(jnp.mean(loss))

            f = tau * (jnp.log(s_b) + cb + rho)
            counts = jnp.sum(oh_g.astype(_f32), axis=1)
            gsum = jnp.sum(jnp.where(oh_g, f[None, :], 0.0), axis=1)
            grad = gsum / jnp.maximum(counts, 1.0)
            z = (1.0 - GAMMA) * z_ref[0, :] + GAMMA * grad
            ghp = -LAMBADA * jnp.log(p + EPS) - LAMBADA
            new_p = p * jnp.exp(2.0 * ETA_P
                                * jnp.clip(z + ghp, -GRAD_CLIP, GRAD_CLIP))
            updates.append(new_p / jnp.sum(new_p))

        loss_ref[0, 0] = means[0] + means[1]
        po_i_ref[0, :] = updates[0]
        po_t_ref[0, :] = updates[1]


def _fused(ids_i, ids_t, img, txt, tau_i, tau_t, so_i, so_t, bo_i, bo_t,
           gid_i, gid_t, p_i, p_t, z_i, z_t):
    full = pl.BlockSpec((1, B), lambda i: (0, 0))
    small = pl.BlockSpec((1, G), lambda i: (0, 0))
    return pl.pallas_call(
        _fused_body,
        grid=(NSTEP + 1,),
        in_specs=[full, full,
                  pl.BlockSpec((R, D), lambda i: (jnp.minimum(i, NSTEP - 1),
                                                  0)),
                  pl.BlockSpec((B, D), lambda i: (0, 0)),
                  full, full, full, full, full, full, full, full,
                  small, small, small, small],
        out_specs=[pl.BlockSpec(memory_space=pltpu.SMEM),
                   pl.BlockSpec((1, G), lambda i: (0, 0)),
                   pl.BlockSpec((1, G), lambda i: (0, 0))],
        out_shape=[jax.ShapeDtypeStruct((1, 1), _f32),
                   jax.ShapeDtypeStruct((1, G), _f32),
                   jax.ShapeDtypeStruct((1, G), _f32)],
        scratch_shapes=[pltpu.VMEM((B, B), _f32),
                        pltpu.VMEM((1, B), _f32),
                        pltpu.VMEM((1, B), _f32),
                        pltpu.VMEM((1, B), _f32)],
    )(ids_i, ids_t, img, txt, tau_i, tau_t, so_i, so_t, bo_i, bo_t,
      gid_i, gid_t, p_i, p_t, z_i, z_t)


def kernel(image_features, text_features, taus_I, taus_T, s_I, s_T, b_I, b_T,
           z_I, z_T, p_I, p_T, group_info_I, group_info_T,
           image_ids, text_ids, epoch, max_epoch):
    tau_i, tau_t, so_i, so_t, bo_i, bo_t, gid_i, gid_t = _sc_gather(
        taus_I, taus_T, s_I, s_T, b_I, b_T,
        group_info_I.astype(_i32), group_info_T.astype(_i32),
        image_ids.astype(_i32), text_ids.astype(_i32))

    row = lambda v: v.reshape(1, -1)
    loss, p_i_new, p_t_new = _fused(
        row(image_ids.astype(_i32)), row(text_ids.astype(_i32)),
        image_features, text_features,
        row(tau_i), row(tau_t), row(so_i), row(so_t),
        row(bo_i), row(bo_t), row(gid_i), row(gid_t),
        row(p_I), row(p_T), row(z_I), row(z_T))
    return loss[0, 0], p_i_new[0, :], p_t_new[0, :]


# speedup vs baseline: 3.3666x; 1.0091x over previous
import functools

import jax
import jax.numpy as jnp
from jax import lax
from jax.experimental import pallas as pl
from jax.experimental.pallas import tpu as pltpu
from jax.experimental.pallas import tpu_sc as plsc

B = 2048
D = 256
G = 8
GAMMA = 0.8
RHO_I = 0.1
RHO_T = 0.1
ETA_P = 0.01
LAMBADA = 0.5
EPS = 1e-10
GRAD_CLIP = 5.0

R = 256
NSTEP = B // R
BM1 = float(B - 1)

_HI = lax.Precision.HIGHEST
_f32 = jnp.float32
_i32 = jnp.int32


SC_CORES = 2
SC_SUBCORES = 16


@functools.cache
def _build_sc_gather():
    nw = SC_CORES * SC_SUBCORES
    bpw = B // nw
    mesh = plsc.VectorSubcoreMesh(core_axis_name="c", subcore_axis_name="s",
                                  num_cores=SC_CORES,
                                  num_subcores=SC_SUBCORES)
    out_type = ([jax.ShapeDtypeStruct((B,), _f32)] * 6
                + [jax.ShapeDtypeStruct((B,), _i32)] * 2)

    @functools.partial(
        pl.kernel, mesh=mesh, out_type=out_type,
        scratch_types=[pltpu.VMEM((bpw,), _i32),
                       pltpu.VMEM((bpw,), _i32)]
        + [pltpu.VMEM((bpw,), _f32)] * 6
        + [pltpu.VMEM((bpw,), _i32)] * 2
        + [pltpu.SemaphoreType.DMA] * 8,
    )
    def sc_gather(taus_i, taus_t, s_i, s_t, b_i, b_t, gi_i, gi_t,
                  ids_i, ids_t,
                  o_tau_i, o_tau_t, o_s_i, o_s_t, o_b_i, o_b_t,
                  o_gi_i, o_gi_t,
                  idx_i_v, idx_t_v, v0, v1, v2, v3, v4, v5, g0, g1,
                  s0, s1, s2, s3, s4, s5, s6, s7):
        wid = lax.axis_index("s") * SC_CORES + lax.axis_index("c")
        sl = pl.ds(wid * bpw, bpw)
        pltpu.sync_copy(ids_i.at[sl], idx_i_v)
        pltpu.sync_copy(ids_t.at[sl], idx_t_v)
        plan = ((taus_i, idx_i_v, v0, s0, o_tau_i),
                (s_i, idx_i_v, v1, s1, o_s_i),
                (b_i, idx_i_v, v2, s2, o_b_i),
                (gi_i, idx_i_v, g0, s6, o_gi_i),
                (taus_t, idx_t_v, v3, s3, o_tau_t),
                (s_t, idx_t_v, v4, s4, o_s_t),
                (b_t, idx_t_v, v5, s5, o_b_t),
                (gi_t, idx_t_v, g1, s7, o_gi_t))
        cps = [pltpu.async_copy(tbl.at[idx], dst, sem)
               for tbl, idx, dst, sem, _ in plan]
        for cp, (_, _, dst, _, out) in zip(cps, plan):
            cp.wait()
            pltpu.sync_copy(dst, out.at[sl])

    return sc_gather


def _sc_gather(*args):
    return _build_sc_gather()(*args)


def _fused_body(ids_i_ref, ids_t_ref, img_ref, txt_ref,
                tau_i_ref, tau_t_ref, so_i_ref, so_t_ref,
                bo_i_ref, bo_t_ref, gid_i_ref, gid_t_ref,
                p_i_ref, p_t_ref, z_i_ref, z_t_ref,
                loss_ref, po_i_ref, po_t_ref,
                sim_ref, rs_ref, cs_ref, d_ref):
    i = pl.program_id(0)

    @pl.when(i < NSTEP)
    def _():
        sim = lax.dot_general(img_ref[...], txt_ref[...],
                              (((1,), (1,)), ((), ())),
                              preferred_element_type=_f32)
        sim_ref[pl.ds(i * R, R), :] = sim
        rs_ref[0, pl.ds(i * R, R)] = jnp.max(sim, axis=1)
        sq = sim_ref[pl.ds(i * R, R), pl.ds(i * R, R)]
        col = lax.broadcasted_iota(_i32, (R, R), 1)
        row = lax.broadcasted_iota(_i32, (R, R), 0)
        d_ref[0, pl.ds(i * R, R)] = jnp.sum(
            jnp.where(col == row, sq, 0.0), axis=1)

        @pl.when(i == 0)
        def _():
            cs_ref[...] = jnp.full((1, B), -jnp.inf, _f32)

        cs_ref[0, :] = jnp.maximum(cs_ref[0, :], jnp.max(sim, axis=0))

    @pl.when(i == NSTEP)
    def _():
        kidx = lax.broadcasted_iota(_i32, (R, B), 1)
        d = d_ref[0, :]
        rtau_i = 1.0 / tau_i_ref[0, :]
        rtau_t = 1.0 / tau_t_ref[0, :]

        def resolve(ids_ref, mx, rtau, bo):
            new_b = jnp.maximum((mx - d) * rtau, bo)
            last_c, cb_c = [], []
            for s in range(NSTEP):
                ids_blk = ids_ref[0, pl.ds(s * R, R)]
                eq = ids_blk[:, None] == ids_ref[0, :][None, :]
                last = jnp.max(jnp.where(eq, kidx, -1), axis=1)
                cb = jnp.sum(jnp.where(kidx == last[:, None],
                                       new_b[None, :], 0.0), axis=1)
                last_c.append(last)
                cb_c.append(cb)
            return jnp.concatenate(last_c), jnp.concatenate(cb_c)

        last_i, cb_i = resolve(ids_i_ref, rs_ref[0, :], rtau_i,
                               bo_i_ref[0, :])
        last_t, cb_t = resolve(ids_t_ref, cs_ref[0, :], rtau_t,
                               bo_t_ref[0, :])

        b_co_i = d * rtau_i + cb_i
        b_co_t = d * rtau_t + cb_t
        sum_i_c, es_i_c = [], []
        sum_t = jnp.zeros((B,), _f32)
        es_t = jnp.zeros((B,), _f32)
        for s in range(NSTEP):
            lo, hi = s * R, (s + 1) * R
            sim = sim_ref[pl.ds(s * R, R), :]
            e_i = jnp.exp(sim * rtau_i[lo:hi][:, None]
                          - b_co_i[lo:hi][:, None])
            sum_i_c.append(jnp.sum(e_i, axis=1))
            es_i_c.append(jnp.sum(e_i * sim, axis=1))
            e_t = jnp.exp(sim * rtau_t[None, :] - b_co_t[None, :])
            sum_t += jnp.sum(e_t, axis=0)
            es_t += jnp.sum(e_t * sim, axis=0)
        sum_i = jnp.concatenate(sum_i_c)
        num_i = jnp.concatenate(es_i_c) - d * sum_i
        num_t = es_t - d * sum_t

        grp = lax.broadcasted_iota(_i32, (G, B), 0)
        means = []
        updates = []
        for (sm, num, cb, last, tau, so_ref, bo_ref, gid_ref, p_ref, z_ref,
             rho) in (
                (sum_i, num_i, cb_i, last_i, tau_i_ref[0, :], so_i_ref,
                 bo_i_ref, gid_i_ref, p_i_ref, z_i_ref, RHO_I),
                (sum_t, num_t, cb_t, last_t, tau_t_ref[0, :], so_t_ref,
                 bo_t_ref, gid_t_ref, p_t_ref, z_t_ref, RHO_T)):
            g = sm / BM1
            s_vals = (1.0 - GAMMA) * so_ref[0, :] * jnp.exp(bo_ref[0, :] - cb) \
                + GAMMA * g
            chunks = []
            for s in range(NSTEP):
                oh = kidx == last[s * R:(s + 1) * R][:, None]
                chunks.append(jnp.sum(jnp.where(oh, s_vals[None, :], 0.0),
                                      axis=1))
            s_b = jnp.concatenate(chunks)

            gid = gid_ref[0, :]
            p = p_ref[0, :]
            oh_g = grp == gid[None, :]
            gw = G * jnp.sum(jnp.where(oh_g, p[:, None], 0.0), axis=0)
            loss = gw * num / BM1 / (s_b + EPS)
            means.append(jnp.mean(loss))

            f = tau * (jnp.log(s_b) + cb + rho)
            counts = jnp.sum(oh_g.astype(_f32), axis=1)
            gsum = jnp.sum(jnp.where(oh_g, f[None, :], 0.0), axis=1)
            grad = gsum / jnp.maximum(counts, 1.0)
            z = (1.0 - GAMMA) * z_ref[0, :] + GAMMA * grad
            ghp = -LAMBADA * jnp.log(p + EPS) - LAMBADA
            new_p = p * jnp.exp(2.0 * ETA_P
                                * jnp.clip(z + ghp, -GRAD_CLIP, GRAD_CLIP))
            updates.append(new_p / jnp.sum(new_p))

        loss_ref[0, 0] = means[0] + means[1]
        po_i_ref[0, :] = updates[0]
        po_t_ref[0, :] = updates[1]


def _fused(ids_i, ids_t, img, txt, tau_i, tau_t, so_i, so_t, bo_i, bo_t,
           gid_i, gid_t, p_i, p_t, z_i, z_t):
    full = pl.BlockSpec((1, B), lambda i: (0, 0))
    small = pl.BlockSpec((1, G), lambda i: (0, 0))
    return pl.pallas_call(
        _fused_body,
        grid=(NSTEP + 1,),
        in_specs=[full, full,
                  pl.BlockSpec((R, D), lambda i: (jnp.minimum(i, NSTEP - 1),
                                                  0)),
                  pl.BlockSpec((B, D), lambda i: (0, 0)),
                  full, full, full, full, full, full, full, full,
                  small, small, small, small],
        out_specs=[pl.BlockSpec(memory_space=pltpu.SMEM),
                   pl.BlockSpec((1, G), lambda i: (0, 0)),
                   pl.BlockSpec((1, G), lambda i: (0, 0))],
        out_shape=[jax.ShapeDtypeStruct((1, 1), _f32),
                   jax.ShapeDtypeStruct((1, G), _f32),
                   jax.ShapeDtypeStruct((1, G), _f32)],
        scratch_shapes=[pltpu.VMEM((B, B), _f32),
                        pltpu.VMEM((1, B), _f32),
                        pltpu.VMEM((1, B), _f32),
                        pltpu.VMEM((1, B), _f32)],
    )(ids_i, ids_t, img, txt, tau_i, tau_t, so_i, so_t, bo_i, bo_t,
      gid_i, gid_t, p_i, p_t, z_i, z_t)


def kernel(image_features, text_features, taus_I, taus_T, s_I, s_T, b_I, b_T,
           z_I, z_T, p_I, p_T, group_info_I, group_info_T,
           image_ids, text_ids, epoch, max_epoch):
    tau_i, tau_t, so_i, so_t, bo_i, bo_t, gid_i, gid_t = _sc_gather(
        taus_I, taus_T, s_I, s_T, b_I, b_T,
        group_info_I.astype(_i32), group_info_T.astype(_i32),
        image_ids.astype(_i32), text_ids.astype(_i32))

    row = lambda v: v.reshape(1, -1)
    loss, p_i_new, p_t_new = _fused(
        row(image_ids.astype(_i32)), row(text_ids.astype(_i32)),
        image_features, text_features,
        row(tau_i), row(tau_t), row(so_i), row(so_t),
        row(bo_i), row(bo_t), row(gid_i), row(gid_t),
        row(p_I), row(p_T), row(z_I), row(z_T))
    return loss[0, 0], p_i_new[0, :], p_t_new[0, :]


# speedup vs baseline: 3.5268x; 1.0476x over previous
import functools

import jax
import jax.numpy as jnp
from jax import lax
from jax.experimental import pallas as pl
from jax.experimental.pallas import tpu as pltpu
from jax.experimental.pallas import tpu_sc as plsc

B = 2048
D = 256
G = 8
GAMMA = 0.8
RHO_I = 0.1
RHO_T = 0.1
ETA_P = 0.01
LAMBADA = 0.5
EPS = 1e-10
GRAD_CLIP = 5.0

R = 512
NSTEP = B // R
BM1 = float(B - 1)

_HI = lax.Precision.HIGHEST
_f32 = jnp.float32
_i32 = jnp.int32


SC_CORES = 2
SC_SUBCORES = 16


@functools.cache
def _build_sc_gather():
    nw = SC_CORES * SC_SUBCORES
    bpw = B // nw
    mesh = plsc.VectorSubcoreMesh(core_axis_name="c", subcore_axis_name="s",
                                  num_cores=SC_CORES,
                                  num_subcores=SC_SUBCORES)
    out_type = ([jax.ShapeDtypeStruct((B,), _f32)] * 6
                + [jax.ShapeDtypeStruct((B,), _i32)] * 2)

    @functools.partial(
        pl.kernel, mesh=mesh, out_type=out_type,
        scratch_types=[pltpu.VMEM((bpw,), _i32),
                       pltpu.VMEM((bpw,), _i32)]
        + [pltpu.VMEM((bpw,), _f32)] * 6
        + [pltpu.VMEM((bpw,), _i32)] * 2
        + [pltpu.SemaphoreType.DMA] * 8,
    )
    def sc_gather(taus_i, taus_t, s_i, s_t, b_i, b_t, gi_i, gi_t,
                  ids_i, ids_t,
                  o_tau_i, o_tau_t, o_s_i, o_s_t, o_b_i, o_b_t,
                  o_gi_i, o_gi_t,
                  idx_i_v, idx_t_v, v0, v1, v2, v3, v4, v5, g0, g1,
                  s0, s1, s2, s3, s4, s5, s6, s7):
        wid = lax.axis_index("s") * SC_CORES + lax.axis_index("c")
        sl = pl.ds(wid * bpw, bpw)
        pltpu.sync_copy(ids_i.at[sl], idx_i_v)
        pltpu.sync_copy(ids_t.at[sl], idx_t_v)
        plan = ((taus_i, idx_i_v, v0, s0, o_tau_i),
                (s_i, idx_i_v, v1, s1, o_s_i),
                (b_i, idx_i_v, v2, s2, o_b_i),
                (gi_i, idx_i_v, g0, s6, o_gi_i),
                (taus_t, idx_t_v, v3, s3, o_tau_t),
                (s_t, idx_t_v, v4, s4, o_s_t),
                (b_t, idx_t_v, v5, s5, o_b_t),
                (gi_t, idx_t_v, g1, s7, o_gi_t))
        cps = [pltpu.async_copy(tbl.at[idx], dst, sem)
               for tbl, idx, dst, sem, _ in plan]
        for cp, (_, _, dst, _, out) in zip(cps, plan):
            cp.wait()
            pltpu.sync_copy(dst, out.at[sl])

    return sc_gather


def _sc_gather(*args):
    return _build_sc_gather()(*args)


def _fused_body(ids_i_ref, ids_t_ref, img_ref, txt_ref,
                tau_i_ref, tau_t_ref, so_i_ref, so_t_ref,
                bo_i_ref, bo_t_ref, gid_i_ref, gid_t_ref,
                p_i_ref, p_t_ref, z_i_ref, z_t_ref,
                loss_ref, po_i_ref, po_t_ref,
                sim_ref, rs_ref, cs_ref, d_ref):
    i = pl.program_id(0)

    @pl.when(i < NSTEP)
    def _():
        sim = lax.dot_general(img_ref[...], txt_ref[...],
                              (((1,), (1,)), ((), ())),
                              preferred_element_type=_f32)
        sim_ref[pl.ds(i * R, R), :] = sim
        rs_ref[0, pl.ds(i * R, R)] = jnp.max(sim, axis=1)
        sq = sim_ref[pl.ds(i * R, R), pl.ds(i * R, R)]
        col = lax.broadcasted_iota(_i32, (R, R), 1)
        row = lax.broadcasted_iota(_i32, (R, R), 0)
        d_ref[0, pl.ds(i * R, R)] = jnp.sum(
            jnp.where(col == row, sq, 0.0), axis=1)

        @pl.when(i == 0)
        def _():
            cs_ref[...] = jnp.full((1, B), -jnp.inf, _f32)

        cs_ref[0, :] = jnp.maximum(cs_ref[0, :], jnp.max(sim, axis=0))

    @pl.when(i == NSTEP)
    def _():
        kidx = lax.broadcasted_iota(_i32, (R, B), 1)
        d = d_ref[0, :]
        rtau_i = 1.0 / tau_i_ref[0, :]
        rtau_t = 1.0 / tau_t_ref[0, :]

        def resolve(ids_ref, mx, rtau, bo):
            new_b = jnp.maximum((mx - d) * rtau, bo)
            last_c, cb_c = [], []
            for s in range(NSTEP):
                ids_blk = ids_ref[0, pl.ds(s * R, R)]
                eq = ids_blk[:, None] == ids_ref[0, :][None, :]
                last = jnp.max(jnp.where(eq, kidx, -1), axis=1)
                cb = jnp.sum(jnp.where(kidx == last[:, None],
                                       new_b[None, :], 0.0), axis=1)
                last_c.append(last)
                cb_c.append(cb)
            return jnp.concatenate(last_c), jnp.concatenate(cb_c)

        last_i, cb_i = resolve(ids_i_ref, rs_ref[0, :], rtau_i,
                               bo_i_ref[0, :])
        last_t, cb_t = resolve(ids_t_ref, cs_ref[0, :], rtau_t,
                               bo_t_ref[0, :])

        b_co_i = d * rtau_i + cb_i
        b_co_t = d * rtau_t + cb_t
        sum_i_c, es_i_c = [], []
        sum_t = jnp.zeros((B,), _f32)
        es_t = jnp.zeros((B,), _f32)
        for s in range(NSTEP):
            lo, hi = s * R, (s + 1) * R
            sim = sim_ref[pl.ds(s * R, R), :]
            e_i = jnp.exp(sim * rtau_i[lo:hi][:, None]
                          - b_co_i[lo:hi][:, None])
            sum_i_c.append(jnp.sum(e_i, axis=1))
            es_i_c.append(jnp.sum(e_i * sim, axis=1))
            e_t = jnp.exp(sim * rtau_t[None, :] - b_co_t[None, :])
            sum_t += jnp.sum(e_t, axis=0)
            es_t += jnp.sum(e_t * sim, axis=0)
        sum_i = jnp.concatenate(sum_i_c)
        num_i = jnp.concatenate(es_i_c) - d * sum_i
        num_t = es_t - d * sum_t

        grp = lax.broadcasted_iota(_i32, (G, B), 0)
        means = []
        updates = []
        for (sm, num, cb, last, tau, so_ref, bo_ref, gid_ref, p_ref, z_ref,
             rho) in (
                (sum_i, num_i, cb_i, last_i, tau_i_ref[0, :], so_i_ref,
                 bo_i_ref, gid_i_ref, p_i_ref, z_i_ref, RHO_I),
                (sum_t, num_t, cb_t, last_t, tau_t_ref[0, :], so_t_ref,
                 bo_t_ref, gid_t_ref, p_t_ref, z_t_ref, RHO_T)):
            g = sm / BM1
            s_vals = (1.0 - GAMMA) * so_ref[0, :] * jnp.exp(bo_ref[0, :] - cb) \
                + GAMMA * g
            chunks = []
            for s in range(NSTEP):
                oh = kidx == last[s * R:(s + 1) * R][:, None]
                chunks.append(jnp.sum(jnp.where(oh, s_vals[None, :], 0.0),
                                      axis=1))
            s_b = jnp.concatenate(chunks)

            gid = gid_ref[0, :]
            p = p_ref[0, :]
            oh_g = grp == gid[None, :]
            gw = G * jnp.sum(jnp.where(oh_g, p[:, None], 0.0), axis=0)
            loss = gw * num / BM1 / (s_b + EPS)
            means.append(jnp.mean(loss))

            f = tau * (jnp.log(s_b) + cb + rho)
            counts = jnp.sum(oh_g.astype(_f32), axis=1)
            gsum = jnp.sum(jnp.where(oh_g, f[None, :], 0.0), axis=1)
            grad = gsum / jnp.maximum(counts, 1.0)
            z = (1.0 - GAMMA) * z_ref[0, :] + GAMMA * grad
            ghp = -LAMBADA * jnp.log(p + EPS) - LAMBADA
            new_p = p * jnp.exp(2.0 * ETA_P
                                * jnp.clip(z + ghp, -GRAD_CLIP, GRAD_CLIP))
            updates.append(new_p / jnp.sum(new_p))

        loss_ref[0, 0] = means[0] + means[1]
        po_i_ref[0, :] = updates[0]
        po_t_ref[0, :] = updates[1]


def _fused(ids_i, ids_t, img, txt, tau_i, tau_t, so_i, so_t, bo_i, bo_t,
           gid_i, gid_t, p_i, p_t, z_i, z_t):
    full = pl.BlockSpec((1, B), lambda i: (0, 0))
    small = pl.BlockSpec((1, G), lambda i: (0, 0))
    return pl.pallas_call(
        _fused_body,
        grid=(NSTEP + 1,),
        in_specs=[full, full,
                  pl.BlockSpec((R, D), lambda i: (jnp.minimum(i, NSTEP - 1),
                                                  0)),
                  pl.BlockSpec((B, D), lambda i: (0, 0)),
                  full, full, full, full, full, full, full, full,
                  small, small, small, small],
        out_specs=[pl.BlockSpec(memory_space=pltpu.SMEM),
                   pl.BlockSpec((1, G), lambda i: (0, 0)),
                   pl.BlockSpec((1, G), lambda i: (0, 0))],
        out_shape=[jax.ShapeDtypeStruct((1, 1), _f32),
                   jax.ShapeDtypeStruct((1, G), _f32),
                   jax.ShapeDtypeStruct((1, G), _f32)],
        scratch_shapes=[pltpu.VMEM((B, B), _f32),
                        pltpu.VMEM((1, B), _f32),
                        pltpu.VMEM((1, B), _f32),
                        pltpu.VMEM((1, B), _f32)],
    )(ids_i, ids_t, img, txt, tau_i, tau_t, so_i, so_t, bo_i, bo_t,
      gid_i, gid_t, p_i, p_t, z_i, z_t)


def kernel(image_features, text_features, taus_I, taus_T, s_I, s_T, b_I, b_T,
           z_I, z_T, p_I, p_T, group_info_I, group_info_T,
           image_ids, text_ids, epoch, max_epoch):
    tau_i, tau_t, so_i, so_t, bo_i, bo_t, gid_i, gid_t = _sc_gather(
        taus_I, taus_T, s_I, s_T, b_I, b_T,
        group_info_I.astype(_i32), group_info_T.astype(_i32),
        image_ids.astype(_i32), text_ids.astype(_i32))

    row = lambda v: v.reshape(1, -1)
    loss, p_i_new, p_t_new = _fused(
        row(image_ids.astype(_i32)), row(text_ids.astype(_i32)),
        image_features, text_features,
        row(tau_i), row(tau_t), row(so_i), row(so_t),
        row(bo_i), row(bo_t), row(gid_i), row(gid_t),
        row(p_I), row(p_T), row(z_I), row(z_T))
    return loss[0, 0], p_i_new[0, :], p_t_new[0, :]


# speedup vs baseline: 3.5371x; 1.0029x over previous
import functools

import jax
import jax.numpy as jnp
from jax import lax
from jax.experimental import pallas as pl
from jax.experimental.pallas import tpu as pltpu
from jax.experimental.pallas import tpu_sc as plsc

B = 2048
D = 256
G = 8
GAMMA = 0.8
RHO_I = 0.1
RHO_T = 0.1
ETA_P = 0.01
LAMBADA = 0.5
EPS = 1e-10
GRAD_CLIP = 5.0

R = 1024
NSTEP = B // R
BM1 = float(B - 1)

_HI = lax.Precision.HIGHEST
_f32 = jnp.float32
_i32 = jnp.int32


SC_CORES = 2
SC_SUBCORES = 16


@functools.cache
def _build_sc_gather():
    nw = SC_CORES * SC_SUBCORES
    bpw = B // nw
    mesh = plsc.VectorSubcoreMesh(core_axis_name="c", subcore_axis_name="s",
                                  num_cores=SC_CORES,
                                  num_subcores=SC_SUBCORES)
    out_type = ([jax.ShapeDtypeStruct((B,), _f32)] * 6
                + [jax.ShapeDtypeStruct((B,), _i32)] * 2)

    @functools.partial(
        pl.kernel, mesh=mesh, out_type=out_type,
        scratch_types=[pltpu.VMEM((bpw,), _i32),
                       pltpu.VMEM((bpw,), _i32)]
        + [pltpu.VMEM((bpw,), _f32)] * 6
        + [pltpu.VMEM((bpw,), _i32)] * 2
        + [pltpu.SemaphoreType.DMA] * 8,
    )
    def sc_gather(taus_i, taus_t, s_i, s_t, b_i, b_t, gi_i, gi_t,
                  ids_i, ids_t,
                  o_tau_i, o_tau_t, o_s_i, o_s_t, o_b_i, o_b_t,
                  o_gi_i, o_gi_t,
                  idx_i_v, idx_t_v, v0, v1, v2, v3, v4, v5, g0, g1,
                  s0, s1, s2, s3, s4, s5, s6, s7):
        wid = lax.axis_index("s") * SC_CORES + lax.axis_index("c")
        sl = pl.ds(wid * bpw, bpw)
        pltpu.sync_copy(ids_i.at[sl], idx_i_v)
        pltpu.sync_copy(ids_t.at[sl], idx_t_v)
        plan = ((taus_i, idx_i_v, v0, s0, o_tau_i),
                (s_i, idx_i_v, v1, s1, o_s_i),
                (b_i, idx_i_v, v2, s2, o_b_i),
                (gi_i, idx_i_v, g0, s6, o_gi_i),
                (taus_t, idx_t_v, v3, s3, o_tau_t),
                (s_t, idx_t_v, v4, s4, o_s_t),
                (b_t, idx_t_v, v5, s5, o_b_t),
                (gi_t, idx_t_v, g1, s7, o_gi_t))
        cps = [pltpu.async_copy(tbl.at[idx], dst, sem)
               for tbl, idx, dst, sem, _ in plan]
        for cp, (_, _, dst, _, out) in zip(cps, plan):
            cp.wait()
            pltpu.sync_copy(dst, out.at[sl])

    return sc_gather


def _sc_gather(*args):
    return _build_sc_gather()(*args)


def _fused_body(ids_i_ref, ids_t_ref, img_ref, txt_ref,
                tau_i_ref, tau_t_ref, so_i_ref, so_t_ref,
                bo_i_ref, bo_t_ref, gid_i_ref, gid_t_ref,
                p_i_ref, p_t_ref, z_i_ref, z_t_ref,
                loss_ref, po_i_ref, po_t_ref,
                sim_ref, rs_ref, cs_ref, d_ref):
    i = pl.program_id(0)

    @pl.when(i < NSTEP)
    def _():
        sim = lax.dot_general(img_ref[...], txt_ref[...],
                              (((1,), (1,)), ((), ())),
                              preferred_element_type=_f32)
        sim_ref[pl.ds(i * R, R), :] = sim
        rs_ref[0, pl.ds(i * R, R)] = jnp.max(sim, axis=1)
        sq = sim_ref[pl.ds(i * R, R), pl.ds(i * R, R)]
        col = lax.broadcasted_iota(_i32, (R, R), 1)
        row = lax.broadcasted_iota(_i32, (R, R), 0)
        d_ref[0, pl.ds(i * R, R)] = jnp.sum(
            jnp.where(col == row, sq, 0.0), axis=1)

        @pl.when(i == 0)
        def _():
            cs_ref[...] = jnp.full((1, B), -jnp.inf, _f32)

        cs_ref[0, :] = jnp.maximum(cs_ref[0, :], jnp.max(sim, axis=0))

    @pl.when(i == NSTEP)
    def _():
        kidx = lax.broadcasted_iota(_i32, (R, B), 1)
        d = d_ref[0, :]
        rtau_i = 1.0 / tau_i_ref[0, :]
        rtau_t = 1.0 / tau_t_ref[0, :]

        def resolve(ids_ref, mx, rtau, bo):
            new_b = jnp.maximum((mx - d) * rtau, bo)
            last_c, cb_c = [], []
            for s in range(NSTEP):
                ids_blk = ids_ref[0, pl.ds(s * R, R)]
                eq = ids_blk[:, None] == ids_ref[0, :][None, :]
                last = jnp.max(jnp.where(eq, kidx, -1), axis=1)
                cb = jnp.sum(jnp.where(kidx == last[:, None],
                                       new_b[None, :], 0.0), axis=1)
                last_c.append(last)
                cb_c.append(cb)
            return jnp.concatenate(last_c), jnp.concatenate(cb_c)

        last_i, cb_i = resolve(ids_i_ref, rs_ref[0, :], rtau_i,
                               bo_i_ref[0, :])
        last_t, cb_t = resolve(ids_t_ref, cs_ref[0, :], rtau_t,
                               bo_t_ref[0, :])

        b_co_i = d * rtau_i + cb_i
        b_co_t = d * rtau_t + cb_t
        sum_i_c, es_i_c = [], []
        sum_t = jnp.zeros((B,), _f32)
        es_t = jnp.zeros((B,), _f32)
        for s in range(NSTEP):
            lo, hi = s * R, (s + 1) * R
            sim = sim_ref[pl.ds(s * R, R), :]
            e_i = jnp.exp(sim * rtau_i[lo:hi][:, None]
                          - b_co_i[lo:hi][:, None])
            sum_i_c.append(jnp.sum(e_i, axis=1))
            es_i_c.append(jnp.sum(e_i * sim, axis=1))
            e_t = jnp.exp(sim * rtau_t[None, :] - b_co_t[None, :])
            sum_t += jnp.sum(e_t, axis=0)
            es_t += jnp.sum(e_t * sim, axis=0)
        sum_i = jnp.concatenate(sum_i_c)
        num_i = jnp.concatenate(es_i_c) - d * sum_i
        num_t = es_t - d * sum_t

        grp = lax.broadcasted_iota(_i32, (G, B), 0)
        means = []
        updates = []
        for (sm, num, cb, last, tau, so_ref, bo_ref, gid_ref, p_ref, z_ref,
             rho) in (
                (sum_i, num_i, cb_i, last_i, tau_i_ref[0, :], so_i_ref,
                 bo_i_ref, gid_i_ref, p_i_ref, z_i_ref, RHO_I),
                (sum_t, num_t, cb_t, last_t, tau_t_ref[0, :], so_t_ref,
                 bo_t_ref, gid_t_ref, p_t_ref, z_t_ref, RHO_T)):
            g = sm / BM1
            s_vals = (1.0 - GAMMA) * so_ref[0, :] * jnp.exp(bo_ref[0, :] - cb) \
                + GAMMA * g
            chunks = []
            for s in range(NSTEP):
                oh = kidx == last[s * R:(s + 1) * R][:, None]
                chunks.append(jnp.sum(jnp.where(oh, s_vals[None, :], 0.0),
                                      axis=1))
            s_b = jnp.concatenate(chunks)

            gid = gid_ref[0, :]
            p = p_ref[0, :]
            oh_g = grp == gid[None, :]
            gw = G * jnp.sum(jnp.where(oh_g, p[:, None], 0.0), axis=0)
            loss = gw * num / BM1 / (s_b + EPS)
            means.append(jnp.mean(loss))

            f = tau * (jnp.log(s_b) + cb + rho)
            counts = jnp.sum(oh_g.astype(_f32), axis=1)
            gsum = jnp.sum(jnp.where(oh_g, f[None, :], 0.0), axis=1)
            grad = gsum / jnp.maximum(counts, 1.0)
            z = (1.0 - GAMMA) * z_ref[0, :] + GAMMA * grad
            ghp = -LAMBADA * jnp.log(p + EPS) - LAMBADA
            new_p = p * jnp.exp(2.0 * ETA_P
                                * jnp.clip(z + ghp, -GRAD_CLIP, GRAD_CLIP))
            updates.append(new_p / jnp.sum(new_p))

        loss_ref[0, 0] = means[0] + means[1]
        po_i_ref[0, :] = updates[0]
        po_t_ref[0, :] = updates[1]


def _fused(ids_i, ids_t, img, txt, tau_i, tau_t, so_i, so_t, bo_i, bo_t,
           gid_i, gid_t, p_i, p_t, z_i, z_t):
    full = pl.BlockSpec((1, B), lambda i: (0, 0))
    small = pl.BlockSpec((1, G), lambda i: (0, 0))
    return pl.pallas_call(
        _fused_body,
        grid=(NSTEP + 1,),
        in_specs=[full, full,
                  pl.BlockSpec((R, D), lambda i: (jnp.minimum(i, NSTEP - 1),
                                                  0)),
                  pl.BlockSpec((B, D), lambda i: (0, 0)),
                  full, full, full, full, full, full, full, full,
                  small, small, small, small],
        out_specs=[pl.BlockSpec(memory_space=pltpu.SMEM),
                   pl.BlockSpec((1, G), lambda i: (0, 0)),
                   pl.BlockSpec((1, G), lambda i: (0, 0))],
        out_shape=[jax.ShapeDtypeStruct((1, 1), _f32),
                   jax.ShapeDtypeStruct((1, G), _f32),
                   jax.ShapeDtypeStruct((1, G), _f32)],
        scratch_shapes=[pltpu.VMEM((B, B), _f32),
                        pltpu.VMEM((1, B), _f32),
                        pltpu.VMEM((1, B), _f32),
                        pltpu.VMEM((1, B), _f32)],
    )(ids_i, ids_t, img, txt, tau_i, tau_t, so_i, so_t, bo_i, bo_t,
      gid_i, gid_t, p_i, p_t, z_i, z_t)


def kernel(image_features, text_features, taus_I, taus_T, s_I, s_T, b_I, b_T,
           z_I, z_T, p_I, p_T, group_info_I, group_info_T,
           image_ids, text_ids, epoch, max_epoch):
    tau_i, tau_t, so_i, so_t, bo_i, bo_t, gid_i, gid_t = _sc_gather(
        taus_I, taus_T, s_I, s_T, b_I, b_T,
        group_info_I.astype(_i32), group_info_T.astype(_i32),
        image_ids.astype(_i32), text_ids.astype(_i32))

    row = lambda v: v.reshape(1, -1)
    loss, p_i_new, p_t_new = _fused(
        row(image_ids.astype(_i32)), row(text_ids.astype(_i32)),
        image_features, text_features,
        row(tau_i), row(tau_t), row(so_i), row(so_t),
        row(bo_i), row(bo_t), row(gid_i), row(gid_t),
        row(p_I), row(p_T), row(z_I), row(z_T))
    return loss[0, 0], p_i_new[0, :], p_t_new[0, :]
